# baseline (speedup 1.0000x reference)
import jax
import jax.numpy as jnp
from jax.experimental import pallas as pl
from jax.experimental.pallas import tpu as pltpu

_BLK = 4096


def _body(zf_ref, x_ref, t_ref, out_ref):
    pid = pl.program_id(0)
    x = x_ref[...]
    t = t_ref[0]
    blk, c = x.shape
    inv_b = 1.0 / (blk * pl.num_programs(0))

    col = jax.lax.broadcasted_iota(jnp.int32, (blk, c), 1)
    xt = jnp.sum(jnp.where(col == t, x, 0.0), axis=1, keepdims=True)
    x0 = x[:, 0:1]

    d = x0 - xt
    sp = jnp.maximum(d, 0.0) + jnp.log(1.0 + jnp.exp(-jnp.abs(d)))

    @pl.when(pid == 0)
    def _():
        out_ref[0, 0] = 0.0

    out_ref[0, 0] += jnp.sum(sp) * inv_b

    @pl.when(zf_ref[pid] != 0)
    def _():
        s = jnp.sum(jnp.exp(x), axis=1, keepdims=True)
        lz = jnp.log(s) - x0
        out_ref[0, 0] += jnp.sum(jnp.where(t == 0, lz - sp, 0.0)) * inv_b


def kernel(inputs, targets):
    B, C = inputs.shape
    grid = B // _BLK
    t32 = targets.astype(jnp.int32)
    t3 = t32.reshape(grid, _BLK, 1)
    zflags = jnp.any(t3 == 0, axis=(1, 2)).astype(jnp.int32)

    out = pl.pallas_call(
        _body,
        grid=(grid,),
        in_specs=[
            pl.BlockSpec((B // _BLK,), lambda i: (0,), memory_space=pltpu.SMEM),
            pl.BlockSpec((_BLK, C), lambda i: (i, 0)),
            pl.BlockSpec((1, _BLK, 1), lambda i: (i, 0, 0)),
        ],
        out_specs=pl.BlockSpec(memory_space=pltpu.SMEM),
        out_shape=jax.ShapeDtypeStruct((1, 1), jnp.float32),
    )(zflags, inputs, t3)
    return out[0, 0]


# speedup vs baseline: 1.0062x; 1.0062x over previous
import jax
import jax.numpy as jnp
from jax.experimental import pallas as pl
from jax.experimental.pallas import tpu as pltpu

_BLK = 2048


def _body(zf_ref, x_ref, t_ref, out_ref):
    pid = pl.program_id(0)
    x = x_ref[...]
    t = t_ref[0]
    blk, c = x.shape
    inv_b = 1.0 / (blk * pl.num_programs(0))

    col = jax.lax.broadcasted_iota(jnp.int32, (blk, c), 1)
    xt = jnp.sum(jnp.where(col == t, x, 0.0), axis=1, keepdims=True)
    x0 = x[:, 0:1]

    d = x0 - xt
    sp = jnp.maximum(d, 0.0) + jnp.log(1.0 + jnp.exp(-jnp.abs(d)))

    @pl.when(pid == 0)
    def _():
        out_ref[0, 0] = 0.0

    out_ref[0, 0] += jnp.sum(sp) * inv_b

    @pl.when(zf_ref[pid] != 0)
    def _():
        s = jnp.sum(jnp.exp(x), axis=1, keepdims=True)
        lz = jnp.log(s) - x0
        out_ref[0, 0] += jnp.sum(jnp.where(t == 0, lz - sp, 0.0)) * inv_b


def kernel(inputs, targets):
    B, C = inputs.shape
    grid = B // _BLK
    t32 = targets.astype(jnp.int32)
    t3 = t32.reshape(grid, _BLK, 1)
    zflags = jnp.any(t3 == 0, axis=(1, 2)).astype(jnp.int32)

    out = pl.pallas_call(
        _body,
        grid=(grid,),
        in_specs=[
            pl.BlockSpec((B // _BLK,), lambda i: (0,), memory_space=pltpu.SMEM),
            pl.BlockSpec((_BLK, C), lambda i: (i, 0)),
            pl.BlockSpec((1, _BLK, 1), lambda i: (i, 0, 0)),
        ],
        out_specs=pl.BlockSpec(memory_space=pltpu.SMEM),
        out_shape=jax.ShapeDtypeStruct((1, 1), jnp.float32),
    )(zflags, inputs, t3)
    return out[0, 0]
